# baseline (speedup 1.0000x reference)
import jax
import jax.numpy as jnp
from jax.experimental import pallas as pl
from jax.experimental.pallas import tpu as pltpu

_TARGET_TILE = 2048


def _pick_tile(hw: int) -> int:
    if hw % 128 != 0:
        return hw
    best = 128
    t = 128
    while t <= min(hw, _TARGET_TILE):
        if hw % t == 0:
            best = t
        t += 128
    return best


def _train_kernel(x_ref, wt_ref, b_ref, lab_ref,
                  plab_ref, lab_part_ref, unlab_part_ref):
    cin = x_ref.shape[1]
    xb = x_ref[0].reshape(cin, -1)
    z = jnp.dot(wt_ref[...], xb,
                preferred_element_type=jnp.float32) + b_ref[...]
    m = jnp.max(z, axis=0, keepdims=True)
    plab = jnp.argmax(z, axis=0)
    plab_ref[0] = plab.reshape(1, -1).astype(jnp.int32)
    s = jnp.sum(jnp.exp(z - m), axis=0, keepdims=True)
    logs = jnp.log(s)
    cls_iota = jax.lax.broadcasted_iota(jnp.int32, z.shape, 0)
    picked = jnp.sum(jnp.where(cls_iota == lab_ref[0], z, 0.0),
                     axis=0, keepdims=True)
    lab_sum = jnp.sum(m + logs - picked)
    unlab_sum = jnp.sum(logs)
    lab_part_ref[...] = jnp.full(lab_part_ref.shape, lab_sum, jnp.float32)
    unlab_part_ref[...] = jnp.full(unlab_part_ref.shape, unlab_sum,
                                   jnp.float32)


def kernel(x, weight, bias, labels):
    B, Cin, H, W = x.shape
    C = weight.shape[1]
    HW = H * W

    w_t = weight.T
    b_col = bias.reshape(C, 1)
    labels3 = labels.reshape(B, 1, HW).astype(jnp.int32)

    plab3, lab_part, unlab_part = pl.pallas_call(
        _train_kernel,
        out_shape=(
            jax.ShapeDtypeStruct((B, 1, HW), jnp.int32),
            jax.ShapeDtypeStruct((B, 1, 1, 128), jnp.float32),
            jax.ShapeDtypeStruct((B, 1, 1, 128), jnp.float32),
        ),
        grid=(B,),
        in_specs=[
            pl.BlockSpec((1, Cin, H, W), lambda b: (b, 0, 0, 0)),
            pl.BlockSpec((C, Cin), lambda b: (0, 0)),
            pl.BlockSpec((C, 1), lambda b: (0, 0)),
            pl.BlockSpec((1, 1, HW), lambda b: (b, 0, 0)),
        ],
        out_specs=(
            pl.BlockSpec((1, 1, HW), lambda b: (b, 0, 0)),
            pl.BlockSpec((1, 1, 1, 128), lambda b: (b, 0, 0, 0)),
            pl.BlockSpec((1, 1, 1, 128), lambda b: (b, 0, 0, 0)),
        ),
        compiler_params=pltpu.CompilerParams(
            dimension_semantics=("parallel",),
        ),
    )(x, w_t, b_col, labels3)

    denom = B * HW
    return (plab3.reshape(B, HW),
            jnp.sum(lab_part[..., 0]) / denom,
            jnp.sum(unlab_part[..., 0]) / denom)


# speedup vs baseline: 3.6740x; 3.6740x over previous
import jax
import jax.numpy as jnp
from jax.experimental import pallas as pl
from jax.experimental.pallas import tpu as pltpu


def _pick_hb(h: int, w: int, target_pixels: int = 2048) -> int:
    best = 1
    for hb in range(1, h + 1):
        if h % hb == 0 and hb * w <= target_pixels:
            best = hb
    return best


def _train_kernel(x_ref, w_ref, b_ref, lab_ref,
                  plab_ref, lab_part_ref, unlab_part_ref):
    hb, w, cin = x_ref.shape[1], x_ref.shape[2], x_ref.shape[3]
    xt = x_ref[0].reshape(hb * w, cin)
    z = jax.lax.dot_general(
        w_ref[...], xt, (((0,), (1,)), ((), ())),
        preferred_element_type=jnp.float32) + b_ref[...]
    m = jnp.max(z, axis=0, keepdims=True)
    plab = jnp.argmax(z, axis=0)
    plab_ref[0] = plab.reshape(1, -1).astype(jnp.int32)
    s = jnp.sum(jnp.exp(z - m), axis=0, keepdims=True)
    logs = jnp.log(s)
    cls_iota = jax.lax.broadcasted_iota(jnp.int32, z.shape, 0)
    picked = jnp.sum(jnp.where(cls_iota == lab_ref[0], z, 0.0),
                     axis=0, keepdims=True)
    lab_sum = jnp.sum(m + logs - picked)
    unlab_sum = jnp.sum(logs)
    lab_part_ref[...] = jnp.full(lab_part_ref.shape, lab_sum, jnp.float32)
    unlab_part_ref[...] = jnp.full(unlab_part_ref.shape, unlab_sum,
                                   jnp.float32)


def kernel(x, weight, bias, labels):
    B, Cin, H, W = x.shape
    C = weight.shape[1]
    HW = H * W
    Hb = _pick_hb(H, W)
    T = Hb * W
    nt = H // Hb

    x_nhwc = jnp.transpose(x, (0, 2, 3, 1))
    b_col = bias.reshape(C, 1)
    labels3 = labels.reshape(B, 1, HW).astype(jnp.int32)

    plab3, lab_part, unlab_part = pl.pallas_call(
        _train_kernel,
        out_shape=(
            jax.ShapeDtypeStruct((B, 1, HW), jnp.int32),
            jax.ShapeDtypeStruct((B, nt, 1, 128), jnp.float32),
            jax.ShapeDtypeStruct((B, nt, 1, 128), jnp.float32),
        ),
        grid=(B, nt),
        in_specs=[
            pl.BlockSpec((1, Hb, W, Cin), lambda b, t: (b, t, 0, 0)),
            pl.BlockSpec((Cin, C), lambda b, t: (0, 0)),
            pl.BlockSpec((C, 1), lambda b, t: (0, 0)),
            pl.BlockSpec((1, 1, T), lambda b, t: (b, 0, t)),
        ],
        out_specs=(
            pl.BlockSpec((1, 1, T), lambda b, t: (b, 0, t)),
            pl.BlockSpec((1, 1, 1, 128), lambda b, t: (b, t, 0, 0)),
            pl.BlockSpec((1, 1, 1, 128), lambda b, t: (b, t, 0, 0)),
        ),
        compiler_params=pltpu.CompilerParams(
            dimension_semantics=("parallel", "parallel"),
        ),
    )(x_nhwc, weight, b_col, labels3)

    denom = B * HW
    return (plab3.reshape(B, HW),
            jnp.sum(lab_part[..., 0]) / denom,
            jnp.sum(unlab_part[..., 0]) / denom)


# speedup vs baseline: 4.4566x; 1.2130x over previous
import jax
import jax.numpy as jnp
from jax.experimental import pallas as pl
from jax.experimental.pallas import tpu as pltpu


def _pick_hb(h: int, w: int, target_pixels: int = 4096) -> int:
    best = 1
    for hb in range(1, h + 1):
        if h % hb == 0 and hb * w <= target_pixels:
            best = hb
    return best


def _train_kernel(x_ref, w_ref, b_ref, lab_ref,
                  plab_ref, lab_part_ref, unlab_part_ref):
    hb, w, cin = x_ref.shape[1], x_ref.shape[2], x_ref.shape[3]
    xt = x_ref[0].reshape(hb * w, cin)
    z = jax.lax.dot_general(
        w_ref[...], xt, (((0,), (1,)), ((), ())),
        preferred_element_type=jnp.float32) + b_ref[...]
    m = jnp.max(z, axis=0, keepdims=True)
    plab = jnp.argmax(z, axis=0)
    plab_ref[0] = plab.reshape(1, -1).astype(jnp.int32)
    s = jnp.sum(jnp.exp(z - m), axis=0, keepdims=True)
    logs = jnp.log(s)
    cls_iota = jax.lax.broadcasted_iota(jnp.int32, z.shape, 0)
    picked = jnp.sum(jnp.where(cls_iota == lab_ref[0], z, 0.0),
                     axis=0, keepdims=True)
    lab_sum = jnp.sum(m + logs - picked)
    unlab_sum = jnp.sum(logs)
    lab_part_ref[...] = jnp.full(lab_part_ref.shape, lab_sum, jnp.float32)
    unlab_part_ref[...] = jnp.full(unlab_part_ref.shape, unlab_sum,
                                   jnp.float32)


def kernel(x, weight, bias, labels):
    B, Cin, H, W = x.shape
    C = weight.shape[1]
    HW = H * W
    Hb = _pick_hb(H, W)
    T = Hb * W
    nt = H // Hb

    x_nhwc = jnp.transpose(x, (0, 2, 3, 1))
    b_col = bias.reshape(C, 1)
    labels3 = labels.reshape(B, 1, HW).astype(jnp.int32)

    plab3, lab_part, unlab_part = pl.pallas_call(
        _train_kernel,
        out_shape=(
            jax.ShapeDtypeStruct((B, 1, HW), jnp.int32),
            jax.ShapeDtypeStruct((B, nt, 1, 128), jnp.float32),
            jax.ShapeDtypeStruct((B, nt, 1, 128), jnp.float32),
        ),
        grid=(B, nt),
        in_specs=[
            pl.BlockSpec((1, Hb, W, Cin), lambda b, t: (b, t, 0, 0)),
            pl.BlockSpec((Cin, C), lambda b, t: (0, 0)),
            pl.BlockSpec((C, 1), lambda b, t: (0, 0)),
            pl.BlockSpec((1, 1, T), lambda b, t: (b, 0, t)),
        ],
        out_specs=(
            pl.BlockSpec((1, 1, T), lambda b, t: (b, 0, t)),
            pl.BlockSpec((1, 1, 1, 128), lambda b, t: (b, t, 0, 0)),
            pl.BlockSpec((1, 1, 1, 128), lambda b, t: (b, t, 0, 0)),
        ),
        compiler_params=pltpu.CompilerParams(
            dimension_semantics=("parallel", "parallel"),
        ),
    )(x_nhwc, weight, b_col, labels3)

    denom = B * HW
    return (plab3.reshape(B, HW),
            jnp.sum(lab_part[..., 0]) / denom,
            jnp.sum(unlab_part[..., 0]) / denom)


# speedup vs baseline: 4.8505x; 1.0884x over previous
import jax
import jax.numpy as jnp
from jax.experimental import pallas as pl
from jax.experimental.pallas import tpu as pltpu


def _train_kernel(x_ref, w_ref, b_ref, lab_ref,
                  plab_ref, lab_part_ref, unlab_part_ref):
    batch, hb, w, cin = x_ref.shape
    t = hb * w
    lab_sum = jnp.float32(0.0)
    unlab_sum = jnp.float32(0.0)
    for bi in range(batch):
        xt = x_ref[bi].reshape(t, cin)
        z = jax.lax.dot_general(
            w_ref[...], xt, (((0,), (1,)), ((), ())),
            preferred_element_type=jnp.float32) + b_ref[...]
        m = jnp.max(z, axis=0, keepdims=True)
        plab_ref[bi] = jnp.argmax(z, axis=0).astype(jnp.int32)
        s = jnp.sum(jnp.exp(z - m), axis=0, keepdims=True)
        logs = jnp.log(s)
        cls_iota = jax.lax.broadcasted_iota(jnp.int32, z.shape, 0)
        labs = lab_ref[bi].reshape(1, t)
        picked = jnp.sum(jnp.where(cls_iota == labs, z, 0.0),
                         axis=0, keepdims=True)
        lab_sum = lab_sum + jnp.sum(m + logs - picked)
        unlab_sum = unlab_sum + jnp.sum(logs)
    lab_part_ref[...] = jnp.full(lab_part_ref.shape, lab_sum, jnp.float32)
    unlab_part_ref[...] = jnp.full(unlab_part_ref.shape, unlab_sum,
                                   jnp.float32)


def kernel(x, weight, bias, labels):
    B, Cin, H, W = x.shape
    C = weight.shape[1]
    HW = H * W
    Hb = 8 if H % 8 == 0 else 1
    T = Hb * W
    nt = H // Hb

    x_nhwc = jnp.transpose(x, (0, 2, 3, 1))
    b_col = bias.reshape(C, 1)
    labels_i = labels.astype(jnp.int32)

    plab, lab_part, unlab_part = pl.pallas_call(
        _train_kernel,
        out_shape=(
            jax.ShapeDtypeStruct((B, HW), jnp.int32),
            jax.ShapeDtypeStruct((nt, 1, 128), jnp.float32),
            jax.ShapeDtypeStruct((nt, 1, 128), jnp.float32),
        ),
        grid=(nt,),
        in_specs=[
            pl.BlockSpec((B, Hb, W, Cin), lambda t: (0, t, 0, 0)),
            pl.BlockSpec((Cin, C), lambda t: (0, 0)),
            pl.BlockSpec((C, 1), lambda t: (0, 0)),
            pl.BlockSpec((B, T), lambda t: (0, t)),
        ],
        out_specs=(
            pl.BlockSpec((B, T), lambda t: (0, t)),
            pl.BlockSpec((1, 1, 128), lambda t: (t, 0, 0)),
            pl.BlockSpec((1, 1, 128), lambda t: (t, 0, 0)),
        ),
        compiler_params=pltpu.CompilerParams(
            dimension_semantics=("arbitrary",),
            vmem_limit_bytes=56 << 20,
        ),
    )(x_nhwc, weight, b_col, labels_i)

    denom = B * HW
    return (plab,
            jnp.sum(lab_part[..., 0]) / denom,
            jnp.sum(unlab_part[..., 0]) / denom)
